# baseline (device time: 56647 ns/iter reference)
import jax
import jax.numpy as jnp
from jax import lax
from jax.experimental import pallas as pl
from jax.experimental.pallas import tpu as pltpu

N_DEV = 4


def kernel(A, B):
    m_per, k = A.shape
    _, n = B.shape

    def body(a_ref, b_ref, out_ref, comm_ref, send_sems, recv_sems):
        my_pos = lax.axis_index("i")
        left = (my_pos - 1) % N_DEV
        right = (my_pos + 1) % N_DEV

        barrier_sem = pltpu.get_barrier_semaphore()
        for nbr in [left, right]:
            pl.semaphore_signal(
                barrier_sem, inc=1,
                device_id=(nbr,), device_id_type=pl.DeviceIdType.MESH,
            )
        pl.semaphore_wait(barrier_sem, 2)

        comm_ref[0, :, :] = a_ref[:, :]
        out_ref[pl.ds(my_pos * m_per, m_per), :] = jnp.dot(
            a_ref[:, :], b_ref[:, :], preferred_element_type=jnp.float32
        )

        for h in range(N_DEV - 1):
            send_slot = h % 2
            recv_slot = (h + 1) % 2
            rdma = pltpu.make_async_remote_copy(
                src_ref=comm_ref.at[send_slot],
                dst_ref=comm_ref.at[recv_slot],
                send_sem=send_sems.at[send_slot],
                recv_sem=recv_sems.at[recv_slot],
                device_id=(right,),
                device_id_type=pl.DeviceIdType.MESH,
            )
            rdma.start()
            rdma.wait()

            origin = (my_pos - h - 1) % N_DEV
            out_ref[pl.ds(origin * m_per, m_per), :] = jnp.dot(
                comm_ref[recv_slot, :, :], b_ref[:, :],
                preferred_element_type=jnp.float32,
            )

    return pl.pallas_call(
        body,
        out_shape=jax.ShapeDtypeStruct((N_DEV * m_per, n), jnp.float32),
        in_specs=[
            pl.BlockSpec(memory_space=pltpu.VMEM),
            pl.BlockSpec(memory_space=pltpu.VMEM),
        ],
        out_specs=pl.BlockSpec(memory_space=pltpu.VMEM),
        scratch_shapes=[
            pltpu.VMEM((2, m_per, k), jnp.bfloat16),
            pltpu.SemaphoreType.DMA((2,)),
            pltpu.SemaphoreType.DMA((2,)),
        ],
        compiler_params=pltpu.CompilerParams(collective_id=0),
    )(A.astype(jnp.bfloat16), B.astype(jnp.bfloat16))


# device time: 33509 ns/iter; 1.6905x vs baseline; 1.6905x over previous
import jax
import jax.numpy as jnp
from jax import lax
from jax.experimental import pallas as pl
from jax.experimental.pallas import tpu as pltpu

N_DEV = 4


def kernel(A, B):
    m_per, k = A.shape
    _, n = B.shape
    half = m_per // 2

    def body(a_ref, b_ref, out_ref, buf_l, buf_r, buf_d,
             send_sems, recv_sems):
        my_pos = lax.axis_index("i")
        left = (my_pos - 1) % N_DEV
        right = (my_pos + 1) % N_DEV

        barrier_sem = pltpu.get_barrier_semaphore()
        for nbr in [left, right]:
            pl.semaphore_signal(
                barrier_sem, inc=1,
                device_id=(nbr,), device_id_type=pl.DeviceIdType.MESH,
            )
        pl.semaphore_wait(barrier_sem, 2)

        h1r = pltpu.make_async_remote_copy(
            src_ref=a_ref, dst_ref=buf_l,
            send_sem=send_sems.at[0], recv_sem=recv_sems.at[0],
            device_id=(right,), device_id_type=pl.DeviceIdType.MESH,
        )
        h1l = pltpu.make_async_remote_copy(
            src_ref=a_ref, dst_ref=buf_r,
            send_sem=send_sems.at[1], recv_sem=recv_sems.at[1],
            device_id=(left,), device_id_type=pl.DeviceIdType.MESH,
        )
        h1r.start()
        h1l.start()

        out_ref[pl.ds(my_pos * m_per, m_per), :] = jnp.dot(
            a_ref[:, :], b_ref[:, :], preferred_element_type=jnp.float32
        )

        h1r.wait_recv()
        h2r = pltpu.make_async_remote_copy(
            src_ref=buf_l.at[pl.ds(0, half)],
            dst_ref=buf_d.at[pl.ds(0, half)],
            send_sem=send_sems.at[2], recv_sem=recv_sems.at[2],
            device_id=(right,), device_id_type=pl.DeviceIdType.MESH,
        )
        h2r.start()

        h1l.wait_recv()
        h2l = pltpu.make_async_remote_copy(
            src_ref=buf_r.at[pl.ds(half, half)],
            dst_ref=buf_d.at[pl.ds(half, half)],
            send_sem=send_sems.at[3], recv_sem=recv_sems.at[3],
            device_id=(left,), device_id_type=pl.DeviceIdType.MESH,
        )
        h2l.start()

        out_ref[pl.ds(left * m_per, m_per), :] = jnp.dot(
            buf_l[:, :], b_ref[:, :], preferred_element_type=jnp.float32
        )
        out_ref[pl.ds(right * m_per, m_per), :] = jnp.dot(
            buf_r[:, :], b_ref[:, :], preferred_element_type=jnp.float32
        )

        diag = (my_pos + 2) % N_DEV
        h2r.wait_recv()
        h2l.wait_recv()
        out_ref[pl.ds(diag * m_per, m_per), :] = jnp.dot(
            buf_d[:, :], b_ref[:, :], preferred_element_type=jnp.float32
        )

        h1r.wait_send()
        h1l.wait_send()
        h2r.wait_send()
        h2l.wait_send()

    return pl.pallas_call(
        body,
        out_shape=jax.ShapeDtypeStruct((N_DEV * m_per, n), jnp.float32),
        in_specs=[
            pl.BlockSpec(memory_space=pltpu.VMEM),
            pl.BlockSpec(memory_space=pltpu.VMEM),
        ],
        out_specs=pl.BlockSpec(memory_space=pltpu.VMEM),
        scratch_shapes=[
            pltpu.VMEM((m_per, k), jnp.bfloat16),
            pltpu.VMEM((m_per, k), jnp.bfloat16),
            pltpu.VMEM((m_per, k), jnp.bfloat16),
            pltpu.SemaphoreType.DMA((4,)),
            pltpu.SemaphoreType.DMA((4,)),
        ],
        compiler_params=pltpu.CompilerParams(collective_id=0),
    )(A.astype(jnp.bfloat16), B.astype(jnp.bfloat16))


# device time: 29559 ns/iter; 1.9164x vs baseline; 1.1336x over previous
import jax
import jax.numpy as jnp
from jax import lax
from jax.experimental import pallas as pl
from jax.experimental.pallas import tpu as pltpu

N_DEV = 4


def kernel(A, B):
    m_per, k = A.shape
    _, n = B.shape
    half = m_per // 2

    def body(a_ref, b_ref, out_ref, a16, b16, buf_l, buf_r, buf_d,
             send_sems, recv_sems):
        my_pos = lax.axis_index("i")
        left = (my_pos - 1) % N_DEV
        right = (my_pos + 1) % N_DEV

        barrier_sem = pltpu.get_barrier_semaphore()
        for nbr in [left, right]:
            pl.semaphore_signal(
                barrier_sem, inc=1,
                device_id=(nbr,), device_id_type=pl.DeviceIdType.MESH,
            )
        pl.semaphore_wait(barrier_sem, 2)

        a16[:, :] = a_ref[:, :].astype(jnp.bfloat16)

        def copy(src, dst, sem, target):
            return pltpu.make_async_remote_copy(
                src_ref=src, dst_ref=dst,
                send_sem=send_sems.at[sem], recv_sem=recv_sems.at[sem],
                device_id=(target,), device_id_type=pl.DeviceIdType.MESH,
            )

        top = pl.ds(0, half)
        bot = pl.ds(half, half)

        h1r_top = copy(a16.at[top], buf_l.at[top], 0, right)
        h1l_bot = copy(a16.at[bot], buf_r.at[bot], 1, left)
        h1r_bot = copy(a16.at[bot], buf_l.at[bot], 2, right)
        h1l_top = copy(a16.at[top], buf_r.at[top], 3, left)
        h1r_top.start()
        h1l_bot.start()
        h1r_bot.start()
        h1l_top.start()

        b16[:, :] = b_ref[:, :].astype(jnp.bfloat16)
        out_ref[pl.ds(my_pos * m_per, m_per), :] = jnp.dot(
            a16[:, :], b16[:, :], preferred_element_type=jnp.float32
        ).astype(jnp.bfloat16)

        h1r_top.wait_recv()
        fwd_r = copy(buf_l.at[top], buf_d.at[top], 4, right)
        fwd_r.start()
        h1l_bot.wait_recv()
        fwd_l = copy(buf_r.at[bot], buf_d.at[bot], 5, left)
        fwd_l.start()

        h1r_bot.wait_recv()
        out_ref[pl.ds(left * m_per, m_per), :] = jnp.dot(
            buf_l[:, :], b16[:, :], preferred_element_type=jnp.float32
        ).astype(jnp.bfloat16)
        h1l_top.wait_recv()
        out_ref[pl.ds(right * m_per, m_per), :] = jnp.dot(
            buf_r[:, :], b16[:, :], preferred_element_type=jnp.float32
        ).astype(jnp.bfloat16)

        diag = (my_pos + 2) % N_DEV
        fwd_r.wait_recv()
        fwd_l.wait_recv()
        out_ref[pl.ds(diag * m_per, m_per), :] = jnp.dot(
            buf_d[:, :], b16[:, :], preferred_element_type=jnp.float32
        ).astype(jnp.bfloat16)

        for rdma in [h1r_top, h1l_bot, h1r_bot, h1l_top, fwd_r, fwd_l]:
            rdma.wait_send()

    return pl.pallas_call(
        body,
        out_shape=jax.ShapeDtypeStruct((N_DEV * m_per, n), jnp.bfloat16),
        in_specs=[
            pl.BlockSpec(memory_space=pltpu.VMEM),
            pl.BlockSpec(memory_space=pltpu.VMEM),
        ],
        out_specs=pl.BlockSpec(memory_space=pltpu.VMEM),
        scratch_shapes=[
            pltpu.VMEM((m_per, k), jnp.bfloat16),
            pltpu.VMEM((k, n), jnp.bfloat16),
            pltpu.VMEM((m_per, k), jnp.bfloat16),
            pltpu.VMEM((m_per, k), jnp.bfloat16),
            pltpu.VMEM((m_per, k), jnp.bfloat16),
            pltpu.SemaphoreType.DMA((6,)),
            pltpu.SemaphoreType.DMA((6,)),
        ],
        compiler_params=pltpu.CompilerParams(collective_id=0),
    )(A, B)
